# baseline (speedup 1.0000x reference)
import functools

import jax
import jax.numpy as jnp
from jax import lax
from jax.experimental import pallas as pl
from jax.experimental.pallas import tpu as pltpu
from jax.experimental.pallas import tpu_sc as plsc

EMB = 64
IMG = 28
BATCH = 16384

_info = plsc.get_sparse_core_info()
_NC = _info.num_cores
_NS = _info.num_subcores
_NW = _NC * _NS
_BPW = BATCH // _NW
_CH = 128
_NCH = _BPW // _CH

_mesh = plsc.VectorSubcoreMesh(core_axis_name="c", subcore_axis_name="s")


@functools.partial(
    pl.kernel,
    mesh=_mesh,
    out_type=jax.ShapeDtypeStruct((BATCH, EMB), jnp.float32),
    scratch_types=[
        pltpu.VMEM((_NCH, _CH), jnp.int32),
        pltpu.VMEM((_BPW, EMB), jnp.float32),
        pltpu.SemaphoreType.DMA,
    ],
    compiler_params=pltpu.CompilerParams(use_tc_tiling_on_sc=False),
)
def _sc_gather(idx_hbm, table_hbm, out_hbm, idx_v, rows_v, sem):
    wid = lax.axis_index("s") * _NC + lax.axis_index("c")
    pltpu.sync_copy(idx_hbm.at[wid], idx_v)
    copies = [
        pltpu.async_copy(
            table_hbm.at[idx_v.at[j]],
            rows_v.at[pl.ds(j * _CH, _CH)],
            sem,
        )
        for j in range(_NCH)
    ]
    for c in copies:
        c.wait()
    pltpu.sync_copy(rows_v, out_hbm.at[pl.ds(wid * _BPW, _BPW)])


_BM = 2048


def _mm_body(emb_ref, w_ref, b_ref, out_ref):
    out_ref[...] = (
        jnp.dot(emb_ref[...], w_ref[...], preferred_element_type=jnp.float32)
        + b_ref[...]
    )


def kernel(x, table, W, b):
    idx = x.astype(jnp.int32).reshape(_NW, _NCH, _CH)
    emb = _sc_gather(idx, table)
    out = pl.pallas_call(
        _mm_body,
        grid=(BATCH // _BM,),
        in_specs=[
            pl.BlockSpec((_BM, EMB), lambda i: (i, 0)),
            pl.BlockSpec((EMB, IMG * IMG), lambda i: (0, 0)),
            pl.BlockSpec((1, IMG * IMG), lambda i: (0, 0)),
        ],
        out_specs=pl.BlockSpec((_BM, IMG * IMG), lambda i: (i, 0)),
        out_shape=jax.ShapeDtypeStruct((BATCH, IMG * IMG), jnp.float32),
    )(emb, W, b.reshape(1, IMG * IMG))
    return out.reshape(-1, IMG, IMG)
